# baseline (speedup 1.0000x reference)
import functools

import jax
import jax.numpy as jnp
from jax.experimental import pallas as pl
from jax.experimental.pallas import tpu as pltpu


def _fused_kernel(adj_ref, x_ref, w1_ref, b1_ref, w2_ref, b2_ref, o_ref,
                  y1_scr, y2_scr, h_scr, *, block_m):
    l = pl.program_id(0)
    i = pl.program_id(1)
    last_i = pl.num_programs(1) - 1

    @pl.when((l == 0) & (i == 0))
    def _():
        y1_scr[...] = jnp.dot(
            x_ref[...], w1_ref[...],
            precision=jax.lax.Precision.DEFAULT,
            preferred_element_type=jnp.float32,
        )

    @pl.when(l == 0)
    def _():
        a = jnp.dot(
            adj_ref[0], y1_scr[...],
            precision=jax.lax.Precision.DEFAULT,
            preferred_element_type=jnp.float32,
        )
        h_scr[pl.ds(i * block_m, block_m), :] = jnp.maximum(a + b1_ref[...], 0.0)

    @pl.when((l == 0) & (i == last_i))
    def _():
        y2_scr[...] = jnp.dot(
            h_scr[...], w2_ref[...],
            precision=jax.lax.Precision.DEFAULT,
            preferred_element_type=jnp.float32,
        )

    @pl.when(l == 1)
    def _():
        a = jnp.dot(
            adj_ref[0], y2_scr[...],
            precision=jax.lax.Precision.DEFAULT,
            preferred_element_type=jnp.float32,
        )
        o_ref[...] = a + b2_ref[...]


def _gcn(x, adj, W1, b1, W2, b2, *, block_m, interpret=False):
    N, F_in = x.shape
    H = W1.shape[1]
    C = W2.shape[1]
    assert N % block_m == 0
    grid = (2, N // block_m)

    return pl.pallas_call(
        functools.partial(_fused_kernel, block_m=block_m),
        grid=grid,
        in_specs=[
            pl.BlockSpec((1, block_m, N), lambda l, i: (l, i, 0)),
            pl.BlockSpec((N, F_in), lambda l, i: (0, 0)),
            pl.BlockSpec((F_in, H), lambda l, i: (0, 0)),
            pl.BlockSpec((1, H), lambda l, i: (0, 0)),
            pl.BlockSpec((H, C), lambda l, i: (0, 0)),
            pl.BlockSpec((1, C), lambda l, i: (0, 0)),
        ],
        out_specs=pl.BlockSpec((block_m, C), lambda l, i: (l * i, 0)),
        out_shape=jax.ShapeDtypeStruct((N, C), jnp.float32),
        scratch_shapes=[
            pltpu.VMEM((N, H), jnp.float32),
            pltpu.VMEM((N, C), jnp.float32),
            pltpu.VMEM((N, H), jnp.float32),
        ],
        compiler_params=pltpu.CompilerParams(
            dimension_semantics=("arbitrary", "arbitrary"),
        ),
        interpret=interpret,
    )(adj, x, W1, b1.reshape(1, H), W2, b2.reshape(1, C))


def kernel(x, adj, W1, b1, W2, b2):
    return _gcn(x, adj, W1, b1, W2, b2, block_m=400)


# speedup vs baseline: 1.0022x; 1.0022x over previous
import functools

import jax
import jax.numpy as jnp
from jax.experimental import pallas as pl
from jax.experimental.pallas import tpu as pltpu


def _fused_kernel(adj_ref, x_ref, w1_ref, b1_ref, w2_ref, b2_ref, o_ref,
                  y1_scr, y2_scr, h_scr, *, block_m):
    l = pl.program_id(0)
    i = pl.program_id(1)
    last_i = pl.num_programs(1) - 1

    @pl.when((l == 0) & (i == 0))
    def _():
        y1_scr[...] = jnp.dot(
            x_ref[...], w1_ref[...],
            precision=jax.lax.Precision.DEFAULT,
            preferred_element_type=jnp.float32,
        ).astype(jnp.bfloat16)

    @pl.when(l == 0)
    def _():
        a = jax.lax.dot_general(
            adj_ref[0], y1_scr[...], (((1,), (0,)), ((), ())),
            precision=jax.lax.Precision.DEFAULT,
            preferred_element_type=jnp.float32,
        )
        h_scr[pl.ds(i * block_m, block_m), :] = jnp.maximum(a + b1_ref[...], 0.0)

    @pl.when((l == 0) & (i == last_i))
    def _():
        y2_scr[...] = jnp.dot(
            h_scr[...], w2_ref[...],
            precision=jax.lax.Precision.DEFAULT,
            preferred_element_type=jnp.float32,
        ).astype(jnp.bfloat16)

    @pl.when(l == 1)
    def _():
        a = jax.lax.dot_general(
            adj_ref[0], y2_scr[...], (((1,), (0,)), ((), ())),
            precision=jax.lax.Precision.DEFAULT,
            preferred_element_type=jnp.float32,
        )
        o_ref[...] = a + b2_ref[...]


def _gcn(x, adj, W1, b1, W2, b2, *, block_m, interpret=False):
    N, F_in = x.shape
    H = W1.shape[1]
    C = W2.shape[1]
    assert N % block_m == 0
    grid = (2, N // block_m)

    return pl.pallas_call(
        functools.partial(_fused_kernel, block_m=block_m),
        grid=grid,
        in_specs=[
            pl.BlockSpec((1, block_m, N), lambda l, i: (l, i, 0)),
            pl.BlockSpec((N, F_in), lambda l, i: (0, 0)),
            pl.BlockSpec((F_in, H), lambda l, i: (0, 0)),
            pl.BlockSpec((1, H), lambda l, i: (0, 0)),
            pl.BlockSpec((H, C), lambda l, i: (0, 0)),
            pl.BlockSpec((1, C), lambda l, i: (0, 0)),
        ],
        out_specs=pl.BlockSpec((block_m, C), lambda l, i: (l * i, 0)),
        out_shape=jax.ShapeDtypeStruct((N, C), jnp.float32),
        scratch_shapes=[
            pltpu.VMEM((N, H), jnp.bfloat16),
            pltpu.VMEM((N, C), jnp.bfloat16),
            pltpu.VMEM((N, H), jnp.float32),
        ],
        compiler_params=pltpu.CompilerParams(
            dimension_semantics=("arbitrary", "arbitrary"),
        ),
        interpret=interpret,
    )(adj, x, W1, b1.reshape(1, H), W2, b2.reshape(1, C))


def kernel(x, adj, W1, b1, W2, b2):
    return _gcn(x, adj, W1, b1, W2, b2, block_m=400)
